# baseline (speedup 1.0000x reference)
import jax
import jax.numpy as jnp
from jax.experimental import pallas as pl


def _add_body(x_ref, p_ref, o_ref):
    o_ref[...] = x_ref[...] + p_ref[...]


def kernel(x, pos_emb):
    B, S, D = x.shape
    S2, D2 = S // 2, D * 2
    x2 = x.reshape(B, S2, D2)
    p2 = pos_emb.reshape(1, S2, D2)
    BB = 128
    out = pl.pallas_call(
        _add_body,
        grid=(B // BB,),
        in_specs=[
            pl.BlockSpec((BB, S2, D2), lambda i: (i, 0, 0)),
            pl.BlockSpec((1, S2, D2), lambda i: (0, 0, 0)),
        ],
        out_specs=pl.BlockSpec((BB, S2, D2), lambda i: (i, 0, 0)),
        out_shape=jax.ShapeDtypeStruct((B, S2, D2), x.dtype),
    )(x2, p2)
    return out.reshape(B, S, D)


# speedup vs baseline: 1.0038x; 1.0038x over previous
import jax
import jax.numpy as jnp
from jax.experimental import pallas as pl


def _add_body(x_ref, p_ref, o_ref):
    o_ref[...] = x_ref[...] + p_ref[...]


def kernel(x, pos_emb):
    B, S, D = x.shape
    S2, D2 = S // 2, D * 2
    x2 = x.reshape(B, S2, D2)
    p2 = pos_emb.reshape(1, S2, D2)
    BB = 256
    out = pl.pallas_call(
        _add_body,
        grid=(B // BB,),
        in_specs=[
            pl.BlockSpec((BB, S2, D2), lambda i: (i, 0, 0)),
            pl.BlockSpec((1, S2, D2), lambda i: (0, 0, 0)),
        ],
        out_specs=pl.BlockSpec((BB, S2, D2), lambda i: (i, 0, 0)),
        out_shape=jax.ShapeDtypeStruct((B, S2, D2), x.dtype),
    )(x2, p2)
    return out.reshape(B, S, D)
